# baseline (speedup 1.0000x reference)
import functools

import jax
import jax.numpy as jnp
from jax import lax
from jax.experimental import pallas as pl
from jax.experimental.pallas import tpu as pltpu
from jax.experimental.pallas import tpu_sc as plsc

_D = 128
_BATCH = 16384

_L = 16
_NW = 32
_B_PER_W = _BATCH // _NW
_CHUNK = 64
_NCHUNK = _B_PER_W // _CHUNK
_NSET = 4
_AHEAD = 2


def _sc_transe(h_hbm, r_hbm, t_hbm, ent_hbm, rel_hbm, out_hbm,
               hi_v, ri_v, ti_v, hr_v, re_v, te_v, rel_sh, hsem, rsem, tsem,
               wsem, isem):
    sid = lax.axis_index("s")
    wid = sid * 2 + lax.axis_index("c")
    base = wid * _B_PER_W
    bs = pl.ds(base, _B_PER_W)
    idx_cp = (pltpu.async_copy(h_hbm.at[bs], hi_v, isem.at[0]),
              pltpu.async_copy(t_hbm.at[bs], ti_v, isem.at[1]),
              pltpu.async_copy(r_hbm.at[bs], ri_v, isem.at[2]))

    @pl.when(sid == 0)
    def _():
        pltpu.sync_copy(rel_hbm, rel_sh)

    idx_cp[0].wait()
    idx_cp[1].wait()

    def issue_h_t(c):
        s = c % _NSET
        ds = pl.ds(c * _CHUNK, _CHUNK)
        return (pltpu.async_copy(ent_hbm.at[hi_v.at[ds]], hr_v.at[s], hsem.at[s]),
                pltpu.async_copy(ent_hbm.at[ti_v.at[ds]], te_v.at[s], tsem.at[s]))

    def issue_r(c):
        s = c % _NSET
        ds = pl.ds(c * _CHUNK, _CHUNK)
        return pltpu.async_copy(rel_sh.at[ri_v.at[ds]], re_v.at[s], rsem.at[s])

    ht = {k: issue_h_t(k) for k in range(min(_AHEAD, _NCHUNK))}
    idx_cp[2].wait()
    plsc.subcore_barrier()
    ra = {k: issue_r(k) for k in range(min(_AHEAD, _NCHUNK))}
    wb = {}
    for c in range(_NCHUNK):
        s = c % _NSET
        cn = c + _AHEAD
        if cn < _NCHUNK:
            if cn - _NSET in wb:
                wb.pop(cn - _NSET).wait()
            ht[cn] = issue_h_t(cn)
            ra[cn] = issue_r(cn)
        cp_h, cp_t = ht.pop(c)
        cp_h.wait()
        ra.pop(c).wait()
        cp_t.wait()

        def body(i, carry):
            for j in range(_D // _L):
                sl = pl.ds(j * _L, _L)
                te_v[s, i, sl] = hr_v[s, i, sl] + re_v[s, i, sl] - te_v[s, i, sl]
            return carry

        lax.fori_loop(0, _CHUNK, body, 0)
        wb[c] = pltpu.async_copy(
            te_v.at[s], out_hbm.at[pl.ds(base + c * _CHUNK, _CHUNK)], wsem.at[s])
    for c in sorted(wb):
        wb[c].wait()


def kernel(h, r, t, entity_emb, relation_emb):
    h = h.astype(jnp.int32)
    r = r.astype(jnp.int32)
    t = t.astype(jnp.int32)
    mesh = plsc.VectorSubcoreMesh(core_axis_name="c", subcore_axis_name="s")
    run = functools.partial(
        pl.kernel,
        mesh=mesh,
        out_type=jax.ShapeDtypeStruct((_BATCH, _D), jnp.float32),
        scratch_types=[
            pltpu.VMEM((_B_PER_W,), jnp.int32),
            pltpu.VMEM((_B_PER_W,), jnp.int32),
            pltpu.VMEM((_B_PER_W,), jnp.int32),
            pltpu.VMEM((_NSET, _CHUNK, _D), jnp.float32),
            pltpu.VMEM((_NSET, _CHUNK, _D), jnp.float32),
            pltpu.VMEM((_NSET, _CHUNK, _D), jnp.float32),
            pltpu.VMEM_SHARED((1000, _D), jnp.float32),
            pltpu.SemaphoreType.DMA((_NSET,)),
            pltpu.SemaphoreType.DMA((_NSET,)),
            pltpu.SemaphoreType.DMA((_NSET,)),
            pltpu.SemaphoreType.DMA((_NSET,)),
            pltpu.SemaphoreType.DMA((3,)),
        ],
    )(_sc_transe)
    return run(h, r, t, entity_emb, relation_emb)


# speedup vs baseline: 1.0113x; 1.0113x over previous
import functools

import jax
import jax.numpy as jnp
from jax import lax
from jax.experimental import pallas as pl
from jax.experimental.pallas import tpu as pltpu
from jax.experimental.pallas import tpu_sc as plsc

_D = 128
_BATCH = 16384

_L = 16
_NW = 32
_B_PER_W = _BATCH // _NW
_CHUNK = 64
_NCHUNK = _B_PER_W // _CHUNK
_NSET = 6
_AHEAD = 3


def _sc_transe(h_hbm, r_hbm, t_hbm, ent_hbm, rel_hbm, out_hbm,
               hi_v, ri_v, ti_v, hr_v, te_v, rel_sh, hsem, rsem, tsem, wsem,
               isem):
    sid = lax.axis_index("s")
    wid = sid * 2 + lax.axis_index("c")
    base = wid * _B_PER_W
    bs = pl.ds(base, _B_PER_W)
    idx_cp = (pltpu.async_copy(h_hbm.at[bs], hi_v, isem.at[0]),
              pltpu.async_copy(t_hbm.at[bs], ti_v, isem.at[1]),
              pltpu.async_copy(r_hbm.at[bs], ri_v, isem.at[2]))

    @pl.when(sid == 0)
    def _():
        pltpu.sync_copy(rel_hbm, rel_sh)

    idx_cp[0].wait()
    idx_cp[1].wait()

    def issue_h_t(c):
        s = c % _NSET
        ds = pl.ds(c * _CHUNK, _CHUNK)
        return (pltpu.async_copy(ent_hbm.at[hi_v.at[ds]], hr_v.at[s], hsem.at[s]),
                pltpu.async_copy(ent_hbm.at[ti_v.at[ds]], te_v.at[s], tsem.at[s]))

    def issue_r_add(c):
        s = c % _NSET
        ds = pl.ds(c * _CHUNK, _CHUNK)
        return pltpu.async_copy(rel_sh.at[ri_v.at[ds]], hr_v.at[s], rsem.at[s],
                                add=True)

    ht = {k: issue_h_t(k) for k in range(min(_AHEAD, _NCHUNK))}
    idx_cp[2].wait()
    plsc.subcore_barrier()
    ht[0][0].wait()
    ra = {0: issue_r_add(0)}
    wb = {}
    for c in range(_NCHUNK):
        s = c % _NSET
        cn = c + _AHEAD
        if cn < _NCHUNK:
            if cn - _NSET in wb:
                wb.pop(cn - _NSET).wait()
            ht[cn] = issue_h_t(cn)
        if c + 1 < _NCHUNK:
            ht[c + 1][0].wait()
            ra[c + 1] = issue_r_add(c + 1)
        ra.pop(c).wait()
        ht.pop(c)[1].wait()

        def body(i, carry):
            for j in range(_D // _L):
                sl = pl.ds(j * _L, _L)
                te_v[s, i, sl] = hr_v[s, i, sl] - te_v[s, i, sl]
            return carry

        lax.fori_loop(0, _CHUNK, body, 0)
        wb[c] = pltpu.async_copy(
            te_v.at[s], out_hbm.at[pl.ds(base + c * _CHUNK, _CHUNK)], wsem.at[s])
    for c in sorted(wb):
        wb[c].wait()


def kernel(h, r, t, entity_emb, relation_emb):
    h = h.astype(jnp.int32)
    r = r.astype(jnp.int32)
    t = t.astype(jnp.int32)
    mesh = plsc.VectorSubcoreMesh(core_axis_name="c", subcore_axis_name="s")
    run = functools.partial(
        pl.kernel,
        mesh=mesh,
        out_type=jax.ShapeDtypeStruct((_BATCH, _D), jnp.float32),
        scratch_types=[
            pltpu.VMEM((_B_PER_W,), jnp.int32),
            pltpu.VMEM((_B_PER_W,), jnp.int32),
            pltpu.VMEM((_B_PER_W,), jnp.int32),
            pltpu.VMEM((_NSET, _CHUNK, _D), jnp.float32),
            pltpu.VMEM((_NSET, _CHUNK, _D), jnp.float32),
            pltpu.VMEM_SHARED((1000, _D), jnp.float32),
            pltpu.SemaphoreType.DMA((_NSET,)),
            pltpu.SemaphoreType.DMA((_NSET,)),
            pltpu.SemaphoreType.DMA((_NSET,)),
            pltpu.SemaphoreType.DMA((_NSET,)),
            pltpu.SemaphoreType.DMA((3,)),
        ],
    )(_sc_transe)
    return run(h, r, t, entity_emb, relation_emb)


# speedup vs baseline: 1.0265x; 1.0150x over previous
import functools

import jax
import jax.numpy as jnp
from jax import lax
from jax.experimental import pallas as pl
from jax.experimental.pallas import tpu as pltpu
from jax.experimental.pallas import tpu_sc as plsc

_D = 128
_BATCH = 16384

_L = 16
_NW = 32
_B_PER_W = _BATCH // _NW
_CHUNK = 64
_NCHUNK = _B_PER_W // _CHUNK
_NSET = 6
_AHEAD = 4


def _sc_transe(h_hbm, r_hbm, t_hbm, ent_hbm, rel_hbm, out_hbm,
               hi_v, ri_v, ti_v, hr_v, te_v, rel_sh, hsem, rsem, tsem, wsem,
               isem):
    sid = lax.axis_index("s")
    wid = sid * 2 + lax.axis_index("c")
    base = wid * _B_PER_W
    bs = pl.ds(base, _B_PER_W)
    idx_cp = (pltpu.async_copy(h_hbm.at[bs], hi_v, isem.at[0]),
              pltpu.async_copy(t_hbm.at[bs], ti_v, isem.at[1]),
              pltpu.async_copy(r_hbm.at[bs], ri_v, isem.at[2]))

    @pl.when(sid == 0)
    def _():
        pltpu.sync_copy(rel_hbm, rel_sh)

    idx_cp[0].wait()
    idx_cp[1].wait()

    def issue_h_t(c):
        s = c % _NSET
        ds = pl.ds(c * _CHUNK, _CHUNK)
        return (pltpu.async_copy(ent_hbm.at[hi_v.at[ds]], hr_v.at[s], hsem.at[s]),
                pltpu.async_copy(ent_hbm.at[ti_v.at[ds]], te_v.at[s], tsem.at[s]))

    def issue_r_add(c):
        s = c % _NSET
        ds = pl.ds(c * _CHUNK, _CHUNK)
        return pltpu.async_copy(rel_sh.at[ri_v.at[ds]], hr_v.at[s], rsem.at[s],
                                add=True)

    ht = {k: issue_h_t(k) for k in range(min(_AHEAD, _NCHUNK))}
    idx_cp[2].wait()
    plsc.subcore_barrier()
    ht[0][0].wait()
    ra = {0: issue_r_add(0)}
    wb = {}
    for c in range(_NCHUNK):
        s = c % _NSET
        cn = c + _AHEAD
        if cn < _NCHUNK:
            if cn - _NSET in wb:
                wb.pop(cn - _NSET).wait()
            ht[cn] = issue_h_t(cn)
        if c + 1 < _NCHUNK:
            ht[c + 1][0].wait()
            ra[c + 1] = issue_r_add(c + 1)
        ra.pop(c).wait()
        ht.pop(c)[1].wait()

        def body(i, carry):
            for j in range(_D // _L):
                sl = pl.ds(j * _L, _L)
                te_v[s, i, sl] = hr_v[s, i, sl] - te_v[s, i, sl]
            return carry

        lax.fori_loop(0, _CHUNK, body, 0)
        wb[c] = pltpu.async_copy(
            te_v.at[s], out_hbm.at[pl.ds(base + c * _CHUNK, _CHUNK)], wsem.at[s])
    for c in sorted(wb):
        wb[c].wait()


def kernel(h, r, t, entity_emb, relation_emb):
    h = h.astype(jnp.int32)
    r = r.astype(jnp.int32)
    t = t.astype(jnp.int32)
    mesh = plsc.VectorSubcoreMesh(core_axis_name="c", subcore_axis_name="s")
    run = functools.partial(
        pl.kernel,
        mesh=mesh,
        out_type=jax.ShapeDtypeStruct((_BATCH, _D), jnp.float32),
        scratch_types=[
            pltpu.VMEM((_B_PER_W,), jnp.int32),
            pltpu.VMEM((_B_PER_W,), jnp.int32),
            pltpu.VMEM((_B_PER_W,), jnp.int32),
            pltpu.VMEM((_NSET, _CHUNK, _D), jnp.float32),
            pltpu.VMEM((_NSET, _CHUNK, _D), jnp.float32),
            pltpu.VMEM_SHARED((1000, _D), jnp.float32),
            pltpu.SemaphoreType.DMA((_NSET,)),
            pltpu.SemaphoreType.DMA((_NSET,)),
            pltpu.SemaphoreType.DMA((_NSET,)),
            pltpu.SemaphoreType.DMA((_NSET,)),
            pltpu.SemaphoreType.DMA((3,)),
        ],
    )(_sc_transe)
    return run(h, r, t, entity_emb, relation_emb)
